# baseline (speedup 1.0000x reference)
import functools

import jax
import jax.numpy as jnp
from jax import lax
from jax.experimental import pallas as pl
from jax.experimental.pallas import tpu as pltpu
from jax.experimental.pallas import tpu_sc as plsc

R = 128
C = 32768
B = 512
NW = 32
ROWS_PER_W = R // NW
E = ROWS_PER_W * B
NCH = 16
CHW = E // NCH
LP = CHW // 16

_mesh = plsc.VectorSubcoreMesh(core_axis_name="c", subcore_axis_name="s")


@functools.partial(
    pl.kernel,
    out_type=jax.ShapeDtypeStruct((NW, 16), jnp.float32),
    mesh=_mesh,
    scratch_types=[
        pltpu.VMEM((NCH, CHW), jnp.int32),
        pltpu.VMEM((NCH, CHW), jnp.int32),
        pltpu.VMEM((NCH, CHW), jnp.float32),
        pltpu.VMEM((NCH, CHW), jnp.float32),
        pltpu.VMEM((16,), jnp.float32),
        pltpu.SemaphoreType.DMA,
    ],
)
def _l1_gather_kernel(out_flat, ind3, tgt3, parts, ind_v, fidx_v, tgt_v,
                      gath_v, acc_v, sem):
    wid = lax.axis_index("s") * 2 + lax.axis_index("c")

    pltpu.sync_copy(ind3.at[wid], ind_v)
    pltpu.sync_copy(tgt3.at[wid], tgt_v)

    row_base = wid * (ROWS_PER_W * C)
    for i in range(NCH):
        off = row_base + (i // 4) * C
        for k in range(LP):
            fidx_v[i, pl.ds(k * 16, 16)] = ind_v[i, pl.ds(k * 16, 16)] + off

    copies = [
        pltpu.make_async_copy(out_flat.at[fidx_v.at[i]], gath_v.at[i], sem)
        for i in range(NCH)
    ]
    for cp in copies:
        cp.start()
    for cp in copies:
        cp.wait()

    acc = jnp.zeros((16,), jnp.float32)
    for i in range(NCH):
        for k in range(LP):
            g = gath_v[i, pl.ds(k * 16, 16)]
            t = tgt_v[i, pl.ds(k * 16, 16)]
            acc = acc + jnp.abs(g - t)

    acc_v[...] = acc
    pltpu.sync_copy(acc_v, parts.at[wid])


def kernel(output, ind, target):
    out_flat = output.reshape(R * C)
    ind3 = ind.astype(jnp.int32).reshape(NW, NCH, CHW)
    tgt3 = target.reshape(NW, NCH, CHW)
    parts = _l1_gather_kernel(out_flat, ind3, tgt3)
    return jnp.sum(parts)


# speedup vs baseline: 1.6264x; 1.6264x over previous
import functools

import jax
import jax.numpy as jnp
from jax import lax
from jax.experimental import pallas as pl
from jax.experimental.pallas import tpu as pltpu
from jax.experimental.pallas import tpu_sc as plsc

R = 128
C = 32768
B = 512
NW = 32
ROWS_PER_W = R // NW
E = ROWS_PER_W * B
NCH = 16
CHW = E // NCH
LP = CHW // 16

_mesh = plsc.VectorSubcoreMesh(core_axis_name="c", subcore_axis_name="s")


@functools.partial(
    pl.kernel,
    out_type=jax.ShapeDtypeStruct((NW, 16), jnp.float32),
    mesh=_mesh,
    scratch_types=[
        pltpu.VMEM((NCH, CHW), jnp.int32),
        pltpu.VMEM((NCH, CHW), jnp.int32),
        pltpu.VMEM((NCH, CHW), jnp.float32),
        pltpu.VMEM((NCH, CHW), jnp.float32),
        pltpu.VMEM((16,), jnp.float32),
        pltpu.SemaphoreType.DMA,
    ],
)
def _l1_gather_kernel(out_flat, ind3, tgt3, parts, ind_v, fidx_v, tgt_v,
                      gath_v, acc_v, sem):
    wid = lax.axis_index("s") * 2 + lax.axis_index("c")

    pltpu.sync_copy(ind3.at[wid], ind_v)
    pltpu.sync_copy(tgt3.at[wid], tgt_v)

    for i in range(NCH):
        row = wid * ROWS_PER_W + (i // 4)
        base = ((row >> 3) << 18) + ((row & 7) << 7)
        for k in range(LP):
            c = ind_v[i, pl.ds(k * 16, 16)]
            fidx_v[i, pl.ds(k * 16, 16)] = (
                base + ((c >> 7) << 10) + (c & 127)
            )

    copies = [
        pltpu.make_async_copy(out_flat.at[fidx_v.at[i]], gath_v.at[i], sem)
        for i in range(NCH)
    ]
    for cp in copies:
        cp.start()
    for cp in copies:
        cp.wait()

    acc = jnp.zeros((16,), jnp.float32)
    for i in range(NCH):
        for k in range(LP):
            g = gath_v[i, pl.ds(k * 16, 16)]
            t = tgt_v[i, pl.ds(k * 16, 16)]
            acc = acc + jnp.abs(g - t)

    acc_v[...] = acc
    pltpu.sync_copy(acc_v, parts.at[wid])


def kernel(output, ind, target):
    out_flat = (
        output.reshape(R // 8, 8, C // 128, 128)
        .transpose(0, 2, 1, 3)
        .reshape(R * C)
    )
    ind3 = ind.astype(jnp.int32).reshape(NW, NCH, CHW)
    tgt3 = target.reshape(NW, NCH, CHW)
    parts = _l1_gather_kernel(out_flat, ind3, tgt3)
    return jnp.sum(parts)


# speedup vs baseline: 1.6684x; 1.0258x over previous
import functools

import jax
import jax.numpy as jnp
from jax import lax
from jax.experimental import pallas as pl
from jax.experimental.pallas import tpu as pltpu
from jax.experimental.pallas import tpu_sc as plsc

R = 128
C = 32768
B = 512
NW = 32
NCH = 16
CHW = 128
LP = CHW // 16

_mesh = plsc.VectorSubcoreMesh(core_axis_name="c", subcore_axis_name="s")


@functools.partial(
    pl.kernel,
    out_type=jax.ShapeDtypeStruct((NW, 16), jnp.float32),
    mesh=_mesh,
    scratch_types=[
        pltpu.VMEM((2, 8, CHW), jnp.int32),
        pltpu.VMEM((NCH, CHW), jnp.int32),
        pltpu.VMEM((2, 8, CHW), jnp.float32),
        pltpu.VMEM((NCH, CHW), jnp.float32),
        pltpu.VMEM((16,), jnp.float32),
        pltpu.SemaphoreType.DMA,
        pltpu.SemaphoreType.DMA,
    ],
)
def _l1_gather_kernel(out_flat, indb, tgtb, parts, ind_v, fidx_v, tgt_v,
                      gath_v, acc_v, insem, gsem):
    wid = lax.axis_index("s") * 2 + lax.axis_index("c")

    cp_ind = pltpu.make_async_copy(indb.at[wid], ind_v, insem)
    cp_tgt = pltpu.make_async_copy(tgtb.at[wid], tgt_v, insem)
    cp_ind.start()
    cp_tgt.start()
    cp_ind.wait()
    cp_tgt.wait()

    row_base = (wid >> 1) << 18
    gathers = []
    for tc in range(2):
        for s in range(8):
            i = tc * 8 + s
            base = row_base + s * 128
            for k in range(LP):
                c = ind_v[tc, s, pl.ds(k * 16, 16)]
                fidx_v[i, pl.ds(k * 16, 16)] = (
                    base + ((c >> 7) << 10) + (c & 127)
                )
            cp = pltpu.make_async_copy(
                out_flat.at[fidx_v.at[i]], gath_v.at[i], gsem)
            cp.start()
            gathers.append(cp)

    acc = jnp.zeros((16,), jnp.float32)
    for tc in range(2):
        for s in range(8):
            i = tc * 8 + s
            gathers[i].wait()
            for k in range(LP):
                g = gath_v[i, pl.ds(k * 16, 16)]
                t = tgt_v[tc, s, pl.ds(k * 16, 16)]
                acc = acc + jnp.abs(g - t)

    acc_v[...] = acc
    pltpu.sync_copy(acc_v, parts.at[wid])


def _tile_major_flat(x):
    r, c = x.shape
    return x.reshape(r // 8, 8, c // 128, 128).transpose(0, 2, 1, 3).reshape(-1)


def kernel(output, ind, target):
    out_flat = _tile_major_flat(output)
    indb = _tile_major_flat(ind.astype(jnp.int32)).reshape(NW, 2, 8, CHW)
    tgtb = _tile_major_flat(target).reshape(NW, 2, 8, CHW)
    parts = _l1_gather_kernel(out_flat, indb, tgtb)
    return jnp.sum(parts)


# speedup vs baseline: 1.7617x; 1.0559x over previous
import functools

import jax
import jax.numpy as jnp
from jax import lax
from jax.experimental import pallas as pl
from jax.experimental.pallas import tpu as pltpu
from jax.experimental.pallas import tpu_sc as plsc

R = 128
C = 32768
B = 512
NW = 32
NCH = 16
CHW = 128
LP = CHW // 16

_mesh = plsc.VectorSubcoreMesh(core_axis_name="c", subcore_axis_name="s")


@functools.partial(
    pl.kernel,
    out_type=jax.ShapeDtypeStruct((NW, 16), jnp.float32),
    mesh=_mesh,
    scratch_types=[
        pltpu.VMEM((NCH, CHW), jnp.int32),
        pltpu.VMEM((NCH, CHW), jnp.float32),
        pltpu.VMEM((NCH, CHW), jnp.float32),
        pltpu.VMEM((16,), jnp.float32),
        pltpu.SemaphoreType.DMA,
        pltpu.SemaphoreType.DMA,
    ],
)
def _l1_gather_kernel(out_flat, indb, tgtb, parts, ind_v, tgt_v,
                      gath_v, acc_v, insem, gsem):
    wid = lax.axis_index("s") * 2 + lax.axis_index("c")

    cp_tgt = pltpu.make_async_copy(tgtb.at[wid], tgt_v, insem)
    cp_tgt.start()
    pltpu.sync_copy(indb.at[wid], ind_v)

    row_base = (wid >> 1) << 18

    def fire_row(i, _):
        base = row_base + ((i & 7) << 7)
        for k in range(LP):
            c = ind_v[i, pl.ds(k * 16, 16)]
            ind_v[i, pl.ds(k * 16, 16)] = base + ((c >> 7) << 10) + (c & 127)
        pltpu.make_async_copy(
            out_flat.at[ind_v.at[i]], gath_v.at[i], gsem).start()
        return 0

    lax.fori_loop(0, NCH, fire_row, 0, unroll=False)

    cp_tgt.wait()

    def drain_row(i, acc):
        pltpu.make_async_copy(
            out_flat.at[ind_v.at[i]], gath_v.at[i], gsem).wait()
        for k in range(LP):
            g = gath_v[i, pl.ds(k * 16, 16)]
            t = tgt_v[i, pl.ds(k * 16, 16)]
            acc = acc + jnp.abs(g - t)
        return acc

    acc = lax.fori_loop(0, NCH, drain_row, jnp.zeros((16,), jnp.float32),
                        unroll=False)

    acc_v[...] = acc
    pltpu.sync_copy(acc_v, parts.at[wid])


def _tile_major_flat(x):
    r, c = x.shape
    return x.reshape(r // 8, 8, c // 128, 128).transpose(0, 2, 1, 3).reshape(-1)


def kernel(output, ind, target):
    out_flat = _tile_major_flat(output)
    indb = _tile_major_flat(ind.astype(jnp.int32)).reshape(NW, NCH, CHW)
    tgtb = _tile_major_flat(target).reshape(NW, NCH, CHW)
    parts = _l1_gather_kernel(out_flat, indb, tgtb)
    return jnp.sum(parts)
